# baseline (speedup 1.0000x reference)
import functools

import jax
import jax.numpy as jnp
from jax import lax
from jax.experimental import pallas as pl
from jax.experimental.pallas import tpu as pltpu
from jax.experimental.pallas import tpu_sc as plsc

_EPS = 1e-5
_F = 128
_RB = 1000
_CB = 40
_NSLOT = 4
_NSUB = 16


def _pre_body(vf, cf, wav, wbv, b1v, wac, wbc, b1c, pv, qc, pc, qv):
    v = vf[...]
    c = cf[...]
    pv[...] = jnp.dot(v, wav[...], preferred_element_type=jnp.float32, precision=lax.Precision.HIGHEST) + b1v[...]
    qc[...] = jnp.dot(c, wbv[...], preferred_element_type=jnp.float32, precision=lax.Precision.HIGHEST)
    pc[...] = jnp.dot(c, wac[...], preferred_element_type=jnp.float32, precision=lax.Precision.HIGHEST) + b1c[...]
    qv[...] = jnp.dot(v, wbc[...], preferred_element_type=jnp.float32, precision=lax.Precision.HIGHEST)


def _pre_call(vf, cf, wav, wbv, b1v, wac, wbc, b1c):
    n = vf.shape[0]
    grid = (n // _RB,)
    blk = pl.BlockSpec((_RB, _F), lambda i: (i, 0))
    wblk = pl.BlockSpec((_F, _F), lambda i: (0, 0))
    bblk = pl.BlockSpec((1, _F), lambda i: (0, 0))
    out = jax.ShapeDtypeStruct((n, _F), jnp.float32)
    return pl.pallas_call(
        _pre_body,
        grid=grid,
        in_specs=[blk, blk, wblk, wblk, bblk, wblk, wblk, bblk],
        out_specs=[blk, blk, blk, blk],
        out_shape=[out, out, out, out],
    )(vf, cf, wav, wbv, b1v[None, :], wac, wbc, b1c[None, :])


def _post1_body(hv, hc, degv, degc, vf, cf,
                w2v, b2v, w2c, b2c,
                uav, ubv, b1uv, w2uv, b2uv,
                uac, ubc, b1uc, w2uc, b2uc,
                uv, sv1, sv2, uc, sc1, sc2):
    aggv = (jnp.dot(hv[...], w2v[...], preferred_element_type=jnp.float32, precision=lax.Precision.HIGHEST)
            + degv[...][:, 0:1] * b2v[...])
    x = (jnp.dot(vf[...], uav[...], preferred_element_type=jnp.float32, precision=lax.Precision.HIGHEST)
         + jnp.dot(aggv, ubv[...], preferred_element_type=jnp.float32, precision=lax.Precision.HIGHEST)
         + b1uv[...])
    u = (jnp.dot(jnp.maximum(x, 0.0), w2uv[...],
                 preferred_element_type=jnp.float32, precision=lax.Precision.HIGHEST) + b2uv[...])
    uv[...] = u
    sv1[...] = jnp.sum(u, axis=0).reshape(1, 1, _F)
    sv2[...] = jnp.sum(u * u, axis=0).reshape(1, 1, _F)

    aggc = (jnp.dot(hc[...], w2c[...], preferred_element_type=jnp.float32, precision=lax.Precision.HIGHEST)
            + degc[...][:, 0:1] * b2c[...])
    y = (jnp.dot(cf[...], uac[...], preferred_element_type=jnp.float32, precision=lax.Precision.HIGHEST)
         + jnp.dot(aggc, ubc[...], preferred_element_type=jnp.float32, precision=lax.Precision.HIGHEST)
         + b1uc[...])
    w = (jnp.dot(jnp.maximum(y, 0.0), w2uc[...],
                 preferred_element_type=jnp.float32, precision=lax.Precision.HIGHEST) + b2uc[...])
    uc[...] = w
    sc1[...] = jnp.sum(w, axis=0).reshape(1, 1, _F)
    sc2[...] = jnp.sum(w * w, axis=0).reshape(1, 1, _F)


def _post1_call(hv, hc, degv, degc, vf, cf, w2v, b2v, w2c, b2c,
                uav, ubv, b1uv, w2uv, b2uv, uac, ubc, b1uc, w2uc, b2uc):
    n = vf.shape[0]
    nb = n // _RB
    grid = (nb,)
    blk = pl.BlockSpec((_RB, _F), lambda i: (i, 0))
    dblk = pl.BlockSpec((_RB, _F), lambda i: (i, 0))
    wblk = pl.BlockSpec((_F, _F), lambda i: (0, 0))
    bblk = pl.BlockSpec((1, _F), lambda i: (0, 0))
    sblk = pl.BlockSpec((1, 1, _F), lambda i: (i, 0, 0))
    outn = jax.ShapeDtypeStruct((n, _F), jnp.float32)
    outs = jax.ShapeDtypeStruct((nb, 1, _F), jnp.float32)
    return pl.pallas_call(
        _post1_body,
        grid=grid,
        in_specs=[blk, blk, dblk, dblk, blk, blk,
                  wblk, bblk, wblk, bblk,
                  wblk, wblk, bblk, wblk, bblk,
                  wblk, wblk, bblk, wblk, bblk],
        out_specs=[blk, sblk, sblk, blk, sblk, sblk],
        out_shape=[outn, outs, outs, outn, outs, outs],
    )(hv, hc, degv, degc, vf, cf,
      w2v, b2v[None, :], w2c, b2c[None, :],
      uav, ubv, b1uv[None, :], w2uv, b2uv[None, :],
      uac, ubc, b1uc[None, :], w2uc, b2uc[None, :])


def _bn_expr(u, s1, s2, g, b, n):
    m = jnp.sum(s1[...], axis=0) / float(n)
    var = jnp.sum(s2[...], axis=0) / float(n) - m * m
    return g[...] * (u[...] - m) * lax.rsqrt(var + _EPS) + b[...]


def _bn2_call(uv, sv1, sv2, vg, vb, uc, sc1, sc2, cg, cb):
    n = uv.shape[0]
    nb = n // _RB

    def body(uv, sv1, sv2, vg, vb, uc, sc1, sc2, cg, cb, ov, oc):
        ov[...] = _bn_expr(uv, sv1, sv2, vg, vb, n)
        oc[...] = _bn_expr(uc, sc1, sc2, cg, cb, n)

    blk = pl.BlockSpec((_RB, _F), lambda i: (i, 0))
    sfull = pl.BlockSpec((nb, 1, _F), lambda i: (0, 0, 0))
    bblk = pl.BlockSpec((1, _F), lambda i: (0, 0))
    out = jax.ShapeDtypeStruct((n, _F), jnp.float32)
    return pl.pallas_call(
        body,
        grid=(nb,),
        in_specs=[blk, sfull, sfull, bblk, bblk, blk, sfull, sfull, bblk, bblk],
        out_specs=[blk, blk],
        out_shape=[out, out],
    )(uv, sv1, sv2, vg[None, :], vb[None, :],
      uc, sc1, sc2, cg[None, :], cb[None, :])


def _bnpre_call(uv, sv1, sv2, vg, vb, uc, sc1, sc2, cg, cb,
                wav, wbv, b1v, wac, wbc, b1c):
    n = uv.shape[0]
    nb = n // _RB

    def body(uv, sv1, sv2, vg, vb, uc, sc1, sc2, cg, cb,
             wav, wbv, b1v, wac, wbc, b1c,
             ov, oc, pv, qc, pc, qv):
        v = _bn_expr(uv, sv1, sv2, vg, vb, n)
        c = _bn_expr(uc, sc1, sc2, cg, cb, n)
        ov[...] = v
        oc[...] = c
        pv[...] = jnp.dot(v, wav[...], preferred_element_type=jnp.float32,
                          precision=lax.Precision.HIGHEST) + b1v[...]
        qc[...] = jnp.dot(c, wbv[...], preferred_element_type=jnp.float32,
                          precision=lax.Precision.HIGHEST)
        pc[...] = jnp.dot(c, wac[...], preferred_element_type=jnp.float32,
                          precision=lax.Precision.HIGHEST) + b1c[...]
        qv[...] = jnp.dot(v, wbc[...], preferred_element_type=jnp.float32,
                          precision=lax.Precision.HIGHEST)

    blk = pl.BlockSpec((_RB, _F), lambda i: (i, 0))
    sfull = pl.BlockSpec((nb, 1, _F), lambda i: (0, 0, 0))
    bblk = pl.BlockSpec((1, _F), lambda i: (0, 0))
    wblk = pl.BlockSpec((_F, _F), lambda i: (0, 0))
    out = jax.ShapeDtypeStruct((n, _F), jnp.float32)
    return pl.pallas_call(
        body,
        grid=(nb,),
        in_specs=[blk, sfull, sfull, bblk, bblk, blk, sfull, sfull, bblk, bblk,
                  wblk, wblk, bblk, wblk, wblk, bblk],
        out_specs=[blk] * 6,
        out_shape=[out] * 6,
    )(uv, sv1, sv2, vg[None, :], vb[None, :],
      uc, sc1, sc2, cg[None, :], cb[None, :],
      wav, wbv, b1v[None, :], wac, wbc, b1c[None, :])


def _zero_rows(buf, nrows):
    z = jnp.zeros((16,), jnp.float32)

    def row(r, carry):
        for j in range(buf.shape[1] // 16):
            buf[r, pl.ds(j * 16, 16)] = z
        return carry

    lax.fori_loop(0, nrows, row, 0)


def _edge_side(p_hbm, q_hbm, ea_hbm, w_hbm, z_hbm, own_hbm, nbr_hbm, out_hbm,
               accum, idx_o, idx_n, buf_p, buf_q, buf_e, wbuf, sem_p, sem_q,
               sem_s, sem_io, sem_in, sem_ea, sid, e_total, n_pad):
    rows_per_sub = n_pad // _NSUB
    pltpu.sync_copy(z_hbm.at[pl.ds(sid * rows_per_sub, rows_per_sub)],
                    accum.at[pl.ds(sid * rows_per_sub, rows_per_sub)])
    pltpu.sync_copy(w_hbm, wbuf)
    plsc.subcore_barrier()

    w_vals = [[wbuf[k, pl.ds(j * 16, 16)] for j in range(_F // 16)]
              for k in range(4)]

    nch = e_total // _CB // _NSUB

    def issue_idx(b, ci):
        base = (ci * _NSUB + sid) * _CB
        pltpu.async_copy(own_hbm.at[pl.ds(base, _CB)], idx_o[b], sem_io[b])
        pltpu.async_copy(nbr_hbm.at[pl.ds(base, _CB)], idx_n[b], sem_in[b])
        pltpu.async_copy(ea_hbm.at[pl.ds(base * 4, _CB * 4)],
                         buf_e[b].at[pl.ds(0, _CB * 4)], sem_ea[b])

    def wait_idx(b):
        base = 0
        pltpu.make_async_copy(own_hbm.at[pl.ds(base, _CB)], idx_o[b],
                              sem_io[b]).wait()
        pltpu.make_async_copy(nbr_hbm.at[pl.ds(base, _CB)], idx_n[b],
                              sem_in[b]).wait()
        pltpu.make_async_copy(ea_hbm.at[pl.ds(base * 4, _CB * 4)],
                              buf_e[b].at[pl.ds(0, _CB * 4)],
                              sem_ea[b]).wait()

    def issue_gather(b):
        pltpu.async_copy(p_hbm.at[idx_o[b]], buf_p[b], sem_p[b])
        pltpu.async_copy(q_hbm.at[idx_n[b]], buf_q[b], sem_q[b])

    def wait_scatter(b):
        pltpu.make_async_copy(buf_p[b], accum.at[idx_o[b]], sem_s[b]).wait()

    def compute(b):
        pltpu.make_async_copy(p_hbm.at[idx_o[b]], buf_p[b], sem_p[b]).wait()
        pltpu.make_async_copy(q_hbm.at[idx_n[b]], buf_q[b], sem_q[b]).wait()
        bp, bq, eb = buf_p[b], buf_q[b], buf_e[b]

        def row(rr, c2):
            av = eb[pl.ds(rr * 8, 16)]
            for h in range(2):
                r = rr * 2 + h
                a0 = av[4 * h]
                a1 = av[4 * h + 1]
                a2 = av[4 * h + 2]
                a3 = av[4 * h + 3]
                for j in range(_F // 16):
                    s = pl.ds(j * 16, 16)
                    acc = bp[r, s] + bq[r, s]
                    acc = acc + a0 * w_vals[0][j]
                    acc = acc + a1 * w_vals[1][j]
                    acc = acc + a2 * w_vals[2][j]
                    acc = acc + a3 * w_vals[3][j]
                    bp[r, s] = jnp.maximum(acc, 0.0)
            return c2

        lax.fori_loop(0, _CB // 2, row, 0)
        pltpu.async_copy(bp, accum.at[idx_o[b]], sem_s[b], add=True)

    issue_idx(0, 0)
    issue_idx(1, 1)
    issue_idx(2, 2)
    wait_idx(0)
    issue_gather(0)
    wait_idx(1)
    issue_gather(1)

    def group(g, carry):
        for b in range(_NSLOT):
            ci = g * _NSLOT + b
            compute(b)
            s3 = (b + 3) % _NSLOT

            @pl.when(jnp.logical_and(ci >= 1, ci + 3 < nch))
            def _():
                wait_scatter(s3)

            @pl.when(ci + 3 < nch)
            def _():
                issue_idx(s3, ci + 3)

            s2 = (b + 2) % _NSLOT

            @pl.when(ci + 2 < nch)
            def _():
                wait_idx(s2)
                issue_gather(s2)
        return carry

    lax.fori_loop(0, nch // _NSLOT, group, 0)
    for b in range(_NSLOT):
        wait_scatter((nch - 4 + b) % _NSLOT)
    plsc.subcore_barrier()
    pltpu.sync_copy(accum.at[pl.ds(sid * rows_per_sub, rows_per_sub)],
                    out_hbm.at[pl.ds(sid * rows_per_sub, rows_per_sub)])


def _edge_call(pv, qc, pc, qv, ea_flat, wv, wc, zeros, src, dst):
    nv = pv.shape[0]
    nc = pc.shape[0]
    n_pad = zeros.shape[0]
    e = src.shape[0]
    mesh = plsc.VectorSubcoreMesh(core_axis_name="c", subcore_axis_name="s")

    ns = _NSLOT

    def body(pv_h, qc_h, pc_h, qv_h, ea_h, wv_h, wc_h, z_h, src_h, dst_h,
             hv_h, hc_h, accum, *scr):
        cid = lax.axis_index("c")
        sid = lax.axis_index("s")
        idx_o = list(scr[0:ns])
        idx_n = list(scr[ns:2 * ns])
        buf_p = list(scr[2 * ns:3 * ns])
        buf_q = list(scr[3 * ns:4 * ns])
        buf_e = list(scr[4 * ns:5 * ns])
        wbuf = scr[5 * ns]
        sems = scr[5 * ns + 1:]
        sem_p = list(sems[0:ns])
        sem_q = list(sems[ns:2 * ns])
        sem_s = list(sems[2 * ns:3 * ns])
        sem_io = list(sems[3 * ns:4 * ns])
        sem_in = list(sems[4 * ns:5 * ns])
        sem_ea = list(sems[5 * ns:6 * ns])

        @pl.when(cid == 0)
        def _():
            _edge_side(pv_h, qc_h, ea_h, wv_h, z_h, src_h, dst_h, hv_h,
                       accum, idx_o, idx_n, buf_p, buf_q, buf_e, wbuf,
                       sem_p, sem_q, sem_s, sem_io, sem_in, sem_ea,
                       sid, e, n_pad)

        @pl.when(cid == 1)
        def _():
            _edge_side(pc_h, qv_h, ea_h, wc_h, z_h, dst_h, src_h, hc_h,
                       accum, idx_o, idx_n, buf_p, buf_q, buf_e, wbuf,
                       sem_p, sem_q, sem_s, sem_io, sem_in, sem_ea,
                       sid, e, n_pad)

    f = pl.kernel(
        body,
        out_type=[jax.ShapeDtypeStruct((n_pad, _F), jnp.float32),
                  jax.ShapeDtypeStruct((n_pad, _F), jnp.float32)],
        mesh=mesh,
        scratch_types=(
            [pltpu.VMEM_SHARED((n_pad, _F), jnp.float32)]
            + [pltpu.VMEM((_CB,), jnp.int32)] * (2 * ns)
            + [pltpu.VMEM((_CB, _F), jnp.float32)] * (2 * ns)
            + [pltpu.VMEM((_CB * 4 + 16,), jnp.float32)] * ns
            + [pltpu.VMEM((4, _F), jnp.float32)]
            + [pltpu.SemaphoreType.DMA] * (6 * ns)
        ),
    )
    return f(pv, qc, pc, qv, ea_flat, wv, wc, zeros, src, dst)


def _degree_call(src, dst, nv, nc):
    e = src.shape[0]
    n_pad = ((max(nv, nc) + _NSUB * 128 - 1) // (_NSUB * 128)) * (_NSUB * 128)
    mesh = plsc.VectorSubcoreMesh(core_axis_name="c", subcore_axis_name="s")

    ns = _NSLOT

    def side(own_hbm, out_hbm, dacc, idx, ones, sem_i, sem_s, sid, n_nodes):
        rows_per_sub = n_nodes // _NSUB
        zrows = ones.shape[0]
        _zero_rows(ones, zrows)
        for k in range(rows_per_sub // zrows):
            pltpu.sync_copy(ones, dacc.at[pl.ds(sid * rows_per_sub + k * zrows,
                                                zrows)])
        one = jnp.ones((16,), jnp.float32)

        def orow(r, c):
            for j in range(ones.shape[1] // 16):
                ones[r, pl.ds(j * 16, 16)] = one
            return c

        lax.fori_loop(0, ones.shape[0], orow, 0)
        plsc.subcore_barrier()

        nch = e // _CB // _NSUB

        def issue_idx(b, ci):
            base = (ci * _NSUB + sid) * _CB
            pltpu.async_copy(own_hbm.at[pl.ds(base, _CB)], idx[b], sem_i[b])

        def wait_idx(b):
            pltpu.make_async_copy(own_hbm.at[pl.ds(0, _CB)], idx[b],
                                  sem_i[b]).wait()

        def wait_scatter(b):
            pltpu.make_async_copy(ones, dacc.at[idx[b]], sem_s[b]).wait()

        issue_idx(0, 0)
        issue_idx(1, 1)
        issue_idx(2, 2)

        def group(g, carry):
            for b in range(ns):
                ci = g * ns + b
                wait_idx(b)
                pltpu.async_copy(ones, dacc.at[idx[b]], sem_s[b], add=True)
                s3 = (b + 3) % ns

                @pl.when(jnp.logical_and(ci >= 1, ci + 3 < nch))
                def _():
                    wait_scatter(s3)

                @pl.when(ci + 3 < nch)
                def _():
                    issue_idx(s3, ci + 3)
            return carry

        lax.fori_loop(0, nch // ns, group, 0)
        for b in range(ns):
            wait_scatter((nch - 4 + b) % ns)
        plsc.subcore_barrier()
        pltpu.sync_copy(dacc.at[pl.ds(sid * rows_per_sub, rows_per_sub)],
                        out_hbm.at[pl.ds(sid * rows_per_sub, rows_per_sub)])

    def body2(src_h, dst_h, dv_h, dc_h, dacc, ones_s, i0, i1, i2, i3,
              si0, si1, si2, si3, ss0, ss1, ss2, ss3):
        cid = lax.axis_index("c")
        sid = lax.axis_index("s")
        idx = [i0, i1, i2, i3]
        sem_i = [si0, si1, si2, si3]
        sem_s = [ss0, ss1, ss2, ss3]

        @pl.when(cid == 0)
        def _():
            side(src_h, dv_h, dacc, idx, ones_s, sem_i, sem_s, sid, n_pad)

        @pl.when(cid == 1)
        def _():
            side(dst_h, dc_h, dacc, idx, ones_s, sem_i, sem_s, sid, n_pad)

    f = pl.kernel(
        body2,
        out_type=[jax.ShapeDtypeStruct((n_pad, _F), jnp.float32),
                  jax.ShapeDtypeStruct((n_pad, _F), jnp.float32)],
        mesh=mesh,
        scratch_types=(
            [pltpu.VMEM_SHARED((n_pad, _F), jnp.float32)]
            + [pltpu.VMEM((_CB, _F), jnp.float32)]
            + [pltpu.VMEM((_CB,), jnp.int32)] * 4
            + [pltpu.SemaphoreType.DMA] * 8
        ),
    )
    return f(src, dst)


def kernel(var_features, cons_features, edge_index, edge_attr,
           vmsg_W1, vmsg_b1, vmsg_W2, vmsg_b2,
           cmsg_W1, cmsg_b1, cmsg_W2, cmsg_b2,
           vupd_W1, vupd_b1, vupd_W2, vupd_b2,
           cupd_W1, cupd_b1, cupd_W2, cupd_b2,
           vbn_gamma, vbn_beta, cbn_gamma, cbn_beta):
    f = var_features.shape[1]
    num_iter = vmsg_W1.shape[0]
    src = edge_index[0]
    dst = edge_index[1]
    nv = var_features.shape[0]
    nc = cons_features.shape[0]
    n_pad = ((max(nv, nc) + _NSUB * 128 - 1) // (_NSUB * 128)) * (_NSUB * 128)

    degv, degc = _degree_call(src, dst, nv, nc)
    ea_flat = edge_attr.reshape(-1)
    zeros = jnp.zeros((n_pad, _F), jnp.float32)

    vf, cf = var_features, cons_features
    pv, qc, pc, qv = _pre_call(
        vf, cf,
        vmsg_W1[0, :f], vmsg_W1[0, f:2 * f], vmsg_b1[0],
        cmsg_W1[0, :f], cmsg_W1[0, f:2 * f], cmsg_b1[0])
    for it in range(num_iter):
        hv, hc = _edge_call(pv, qc, pc, qv, ea_flat,
                            vmsg_W1[it, 2 * f:], cmsg_W1[it, 2 * f:],
                            zeros, src, dst)
        uv, sv1, sv2, uc, sc1, sc2 = _post1_call(
            hv, hc, degv, degc, vf, cf,
            vmsg_W2[it], vmsg_b2[it], cmsg_W2[it], cmsg_b2[it],
            vupd_W1[it, :f], vupd_W1[it, f:], vupd_b1[it],
            vupd_W2[it], vupd_b2[it],
            cupd_W1[it, :f], cupd_W1[it, f:], cupd_b1[it],
            cupd_W2[it], cupd_b2[it])
        if it + 1 < num_iter:
            nx = it + 1
            vf, cf, pv, qc, pc, qv = _bnpre_call(
                uv, sv1, sv2, vbn_gamma[it], vbn_beta[it],
                uc, sc1, sc2, cbn_gamma[it], cbn_beta[it],
                vmsg_W1[nx, :f], vmsg_W1[nx, f:2 * f], vmsg_b1[nx],
                cmsg_W1[nx, :f], cmsg_W1[nx, f:2 * f], cmsg_b1[nx])
        else:
            vf, cf = _bn2_call(
                uv, sv1, sv2, vbn_gamma[it], vbn_beta[it],
                uc, sc1, sc2, cbn_gamma[it], cbn_beta[it])
    return vf, cf


# speedup vs baseline: 1.0037x; 1.0037x over previous
import jax
import jax.numpy as jnp
from jax import lax
from jax.experimental import pallas as pl
from jax.experimental.pallas import tpu as pltpu
from jax.experimental.pallas import tpu_sc as plsc

_EPS = 1e-5
_F = 128
_RB = 1000
_CB = 40
_NSLOT = 4
_NSUB = 16


def _pre_body(vf, cf, wav, wbv, b1v, wac, wbc, b1c, pv, qc, pc, qv):
    v = vf[...]
    c = cf[...]
    pv[...] = jnp.dot(v, wav[...], preferred_element_type=jnp.float32, precision=lax.Precision.HIGHEST) + b1v[...]
    qc[...] = jnp.dot(c, wbv[...], preferred_element_type=jnp.float32, precision=lax.Precision.HIGHEST)
    pc[...] = jnp.dot(c, wac[...], preferred_element_type=jnp.float32, precision=lax.Precision.HIGHEST) + b1c[...]
    qv[...] = jnp.dot(v, wbc[...], preferred_element_type=jnp.float32, precision=lax.Precision.HIGHEST)


def _pre_call(vf, cf, wav, wbv, b1v, wac, wbc, b1c):
    n = vf.shape[0]
    grid = (n // _RB,)
    blk = pl.BlockSpec((_RB, _F), lambda i: (i, 0))
    wblk = pl.BlockSpec((_F, _F), lambda i: (0, 0))
    bblk = pl.BlockSpec((1, _F), lambda i: (0, 0))
    out = jax.ShapeDtypeStruct((n, _F), jnp.float32)
    return pl.pallas_call(
        _pre_body,
        grid=grid,
        in_specs=[blk, blk, wblk, wblk, bblk, wblk, wblk, bblk],
        out_specs=[blk, blk, blk, blk],
        out_shape=[out, out, out, out],
    )(vf, cf, wav, wbv, b1v[None, :], wac, wbc, b1c[None, :])


def _post1_body(hv, hc, degv, degc, vf, cf,
                w2v, b2v, w2c, b2c,
                uav, ubv, b1uv, w2uv, b2uv,
                uac, ubc, b1uc, w2uc, b2uc,
                uv, sv1, sv2, uc, sc1, sc2):
    aggv = (jnp.dot(hv[...], w2v[...], preferred_element_type=jnp.float32, precision=lax.Precision.HIGHEST)
            + degv[...][:, 0:1] * b2v[...])
    x = (jnp.dot(vf[...], uav[...], preferred_element_type=jnp.float32, precision=lax.Precision.HIGHEST)
         + jnp.dot(aggv, ubv[...], preferred_element_type=jnp.float32, precision=lax.Precision.HIGHEST)
         + b1uv[...])
    u = (jnp.dot(jnp.maximum(x, 0.0), w2uv[...],
                 preferred_element_type=jnp.float32, precision=lax.Precision.HIGHEST) + b2uv[...])
    uv[...] = u
    sv1[...] = jnp.sum(u, axis=0).reshape(1, 1, _F)
    sv2[...] = jnp.sum(u * u, axis=0).reshape(1, 1, _F)

    aggc = (jnp.dot(hc[...], w2c[...], preferred_element_type=jnp.float32, precision=lax.Precision.HIGHEST)
            + degc[...][:, 0:1] * b2c[...])
    y = (jnp.dot(cf[...], uac[...], preferred_element_type=jnp.float32, precision=lax.Precision.HIGHEST)
         + jnp.dot(aggc, ubc[...], preferred_element_type=jnp.float32, precision=lax.Precision.HIGHEST)
         + b1uc[...])
    w = (jnp.dot(jnp.maximum(y, 0.0), w2uc[...],
                 preferred_element_type=jnp.float32, precision=lax.Precision.HIGHEST) + b2uc[...])
    uc[...] = w
    sc1[...] = jnp.sum(w, axis=0).reshape(1, 1, _F)
    sc2[...] = jnp.sum(w * w, axis=0).reshape(1, 1, _F)


def _post1_call(hv, hc, degv, degc, vf, cf, w2v, b2v, w2c, b2c,
                uav, ubv, b1uv, w2uv, b2uv, uac, ubc, b1uc, w2uc, b2uc):
    n = vf.shape[0]
    nb = n // _RB
    grid = (nb,)
    blk = pl.BlockSpec((_RB, _F), lambda i: (i, 0))
    dblk = pl.BlockSpec((_RB, _F), lambda i: (i, 0))
    wblk = pl.BlockSpec((_F, _F), lambda i: (0, 0))
    bblk = pl.BlockSpec((1, _F), lambda i: (0, 0))
    sblk = pl.BlockSpec((1, 1, _F), lambda i: (i, 0, 0))
    outn = jax.ShapeDtypeStruct((n, _F), jnp.float32)
    outs = jax.ShapeDtypeStruct((nb, 1, _F), jnp.float32)
    return pl.pallas_call(
        _post1_body,
        grid=grid,
        in_specs=[blk, blk, dblk, dblk, blk, blk,
                  wblk, bblk, wblk, bblk,
                  wblk, wblk, bblk, wblk, bblk,
                  wblk, wblk, bblk, wblk, bblk],
        out_specs=[blk, sblk, sblk, blk, sblk, sblk],
        out_shape=[outn, outs, outs, outn, outs, outs],
    )(hv, hc, degv, degc, vf, cf,
      w2v, b2v[None, :], w2c, b2c[None, :],
      uav, ubv, b1uv[None, :], w2uv, b2uv[None, :],
      uac, ubc, b1uc[None, :], w2uc, b2uc[None, :])


def _bn_expr(u, s1, s2, g, b, n):
    m = jnp.sum(s1[...], axis=0) / float(n)
    var = jnp.sum(s2[...], axis=0) / float(n) - m * m
    return g[...] * (u[...] - m) * lax.rsqrt(var + _EPS) + b[...]


def _bn2_call(uv, sv1, sv2, vg, vb, uc, sc1, sc2, cg, cb):
    n = uv.shape[0]
    nb = n // _RB

    def body(uv, sv1, sv2, vg, vb, uc, sc1, sc2, cg, cb, ov, oc):
        ov[...] = _bn_expr(uv, sv1, sv2, vg, vb, n)
        oc[...] = _bn_expr(uc, sc1, sc2, cg, cb, n)

    blk = pl.BlockSpec((_RB, _F), lambda i: (i, 0))
    sfull = pl.BlockSpec((nb, 1, _F), lambda i: (0, 0, 0))
    bblk = pl.BlockSpec((1, _F), lambda i: (0, 0))
    out = jax.ShapeDtypeStruct((n, _F), jnp.float32)
    return pl.pallas_call(
        body,
        grid=(nb,),
        in_specs=[blk, sfull, sfull, bblk, bblk, blk, sfull, sfull, bblk, bblk],
        out_specs=[blk, blk],
        out_shape=[out, out],
    )(uv, sv1, sv2, vg[None, :], vb[None, :],
      uc, sc1, sc2, cg[None, :], cb[None, :])


def _bnpre_call(uv, sv1, sv2, vg, vb, uc, sc1, sc2, cg, cb,
                wav, wbv, b1v, wac, wbc, b1c):
    n = uv.shape[0]
    nb = n // _RB

    def body(uv, sv1, sv2, vg, vb, uc, sc1, sc2, cg, cb,
             wav, wbv, b1v, wac, wbc, b1c,
             ov, oc, pv, qc, pc, qv):
        v = _bn_expr(uv, sv1, sv2, vg, vb, n)
        c = _bn_expr(uc, sc1, sc2, cg, cb, n)
        ov[...] = v
        oc[...] = c
        pv[...] = jnp.dot(v, wav[...], preferred_element_type=jnp.float32,
                          precision=lax.Precision.HIGHEST) + b1v[...]
        qc[...] = jnp.dot(c, wbv[...], preferred_element_type=jnp.float32,
                          precision=lax.Precision.HIGHEST)
        pc[...] = jnp.dot(c, wac[...], preferred_element_type=jnp.float32,
                          precision=lax.Precision.HIGHEST) + b1c[...]
        qv[...] = jnp.dot(v, wbc[...], preferred_element_type=jnp.float32,
                          precision=lax.Precision.HIGHEST)

    blk = pl.BlockSpec((_RB, _F), lambda i: (i, 0))
    sfull = pl.BlockSpec((nb, 1, _F), lambda i: (0, 0, 0))
    bblk = pl.BlockSpec((1, _F), lambda i: (0, 0))
    wblk = pl.BlockSpec((_F, _F), lambda i: (0, 0))
    out = jax.ShapeDtypeStruct((n, _F), jnp.float32)
    return pl.pallas_call(
        body,
        grid=(nb,),
        in_specs=[blk, sfull, sfull, bblk, bblk, blk, sfull, sfull, bblk, bblk,
                  wblk, wblk, bblk, wblk, wblk, bblk],
        out_specs=[blk] * 6,
        out_shape=[out] * 6,
    )(uv, sv1, sv2, vg[None, :], vb[None, :],
      uc, sc1, sc2, cg[None, :], cb[None, :],
      wav, wbv, b1v[None, :], wac, wbc, b1c[None, :])


def _edge_side(p_hbm, q_hbm, ea_hbm, w_hbm, z_hbm, own_hbm, nbr_hbm, out_hbm,
               accum, idx_o, idx_n, buf_p, buf_q, buf_e, wbuf, sem_p, sem_q,
               sem_s, sem_io, sem_in, sem_ea, sid, e_total, n_pad,
               deg_out_hbm=None, ones=None):
    rows_per_sub = n_pad // _NSUB
    rsl = pl.ds(sid * rows_per_sub, rows_per_sub)
    pltpu.sync_copy(z_hbm.at[rsl], accum.at[rsl])
    pltpu.sync_copy(w_hbm, wbuf)
    plsc.subcore_barrier()

    if deg_out_hbm is not None:
        nch_d = e_total // _CB // _NSUB
        one = jnp.ones((16,), jnp.float32)

        def orow(r, c):
            for j in range(_F // 16):
                ones[r, pl.ds(j * 16, 16)] = one
            return c

        lax.fori_loop(0, _CB, orow, 0)

        def d_issue(b, ci):
            base = (ci * _NSUB + sid) * _CB
            pltpu.async_copy(own_hbm.at[pl.ds(base, _CB)], idx_o[b],
                             sem_io[b])

        def d_wait_idx(b):
            pltpu.make_async_copy(own_hbm.at[pl.ds(0, _CB)], idx_o[b],
                                  sem_io[b]).wait()

        def d_wait_sc(b):
            pltpu.make_async_copy(ones, accum.at[idx_o[b]], sem_s[b]).wait()

        d_issue(0, 0)
        d_issue(1, 1)
        d_issue(2, 2)

        def dgroup(g, carry):
            for b in range(_NSLOT):
                ci = g * _NSLOT + b
                d_wait_idx(b)
                pltpu.async_copy(ones, accum.at[idx_o[b]], sem_s[b],
                                 add=True)
                s3 = (b + 3) % _NSLOT

                @pl.when(jnp.logical_and(ci >= 1, ci + 3 < nch_d))
                def _():
                    d_wait_sc(s3)

                @pl.when(ci + 3 < nch_d)
                def _():
                    d_issue(s3, ci + 3)
            return carry

        lax.fori_loop(0, nch_d // _NSLOT, dgroup, 0)
        for b in range(_NSLOT):
            d_wait_sc((nch_d - 4 + b) % _NSLOT)
        plsc.subcore_barrier()
        pltpu.sync_copy(accum.at[rsl], deg_out_hbm.at[rsl])
        pltpu.sync_copy(z_hbm.at[rsl], accum.at[rsl])
        plsc.subcore_barrier()

    w_vals = [[wbuf[k, pl.ds(j * 16, 16)] for j in range(_F // 16)]
              for k in range(4)]

    nch = e_total // _CB // _NSUB

    def issue_idx(b, ci):
        base = (ci * _NSUB + sid) * _CB
        pltpu.async_copy(own_hbm.at[pl.ds(base, _CB)], idx_o[b], sem_io[b])
        pltpu.async_copy(nbr_hbm.at[pl.ds(base, _CB)], idx_n[b], sem_in[b])
        pltpu.async_copy(ea_hbm.at[pl.ds(base * 4, _CB * 4)],
                         buf_e[b].at[pl.ds(0, _CB * 4)], sem_ea[b])

    def wait_idx(b):
        base = 0
        pltpu.make_async_copy(own_hbm.at[pl.ds(base, _CB)], idx_o[b],
                              sem_io[b]).wait()
        pltpu.make_async_copy(nbr_hbm.at[pl.ds(base, _CB)], idx_n[b],
                              sem_in[b]).wait()
        pltpu.make_async_copy(ea_hbm.at[pl.ds(base * 4, _CB * 4)],
                              buf_e[b].at[pl.ds(0, _CB * 4)],
                              sem_ea[b]).wait()

    def issue_gather(b):
        pltpu.async_copy(p_hbm.at[idx_o[b]], buf_p[b], sem_p[b])
        pltpu.async_copy(q_hbm.at[idx_n[b]], buf_q[b], sem_q[b])

    def wait_scatter(b):
        pltpu.make_async_copy(buf_p[b], accum.at[idx_o[b]], sem_s[b]).wait()

    def compute(b):
        pltpu.make_async_copy(p_hbm.at[idx_o[b]], buf_p[b], sem_p[b]).wait()
        pltpu.make_async_copy(q_hbm.at[idx_n[b]], buf_q[b], sem_q[b]).wait()
        bp, bq, eb = buf_p[b], buf_q[b], buf_e[b]

        def row(rr, c2):
            av = eb[pl.ds(rr * 8, 16)]
            for h in range(2):
                r = rr * 2 + h
                a0 = av[4 * h]
                a1 = av[4 * h + 1]
                a2 = av[4 * h + 2]
                a3 = av[4 * h + 3]
                for j in range(_F // 16):
                    s = pl.ds(j * 16, 16)
                    acc = bp[r, s] + bq[r, s]
                    acc = acc + a0 * w_vals[0][j]
                    acc = acc + a1 * w_vals[1][j]
                    acc = acc + a2 * w_vals[2][j]
                    acc = acc + a3 * w_vals[3][j]
                    bp[r, s] = jnp.maximum(acc, 0.0)
            return c2

        lax.fori_loop(0, _CB // 2, row, 0)
        pltpu.async_copy(bp, accum.at[idx_o[b]], sem_s[b], add=True)

    issue_idx(0, 0)
    issue_idx(1, 1)
    issue_idx(2, 2)
    wait_idx(0)
    issue_gather(0)
    wait_idx(1)
    issue_gather(1)

    def group(g, carry):
        for b in range(_NSLOT):
            ci = g * _NSLOT + b
            compute(b)
            s3 = (b + 3) % _NSLOT

            @pl.when(jnp.logical_and(ci >= 1, ci + 3 < nch))
            def _():
                wait_scatter(s3)

            @pl.when(ci + 3 < nch)
            def _():
                issue_idx(s3, ci + 3)

            s2 = (b + 2) % _NSLOT

            @pl.when(ci + 2 < nch)
            def _():
                wait_idx(s2)
                issue_gather(s2)
        return carry

    lax.fori_loop(0, nch // _NSLOT, group, 0)
    for b in range(_NSLOT):
        wait_scatter((nch - 4 + b) % _NSLOT)
    plsc.subcore_barrier()
    pltpu.sync_copy(accum.at[pl.ds(sid * rows_per_sub, rows_per_sub)],
                    out_hbm.at[pl.ds(sid * rows_per_sub, rows_per_sub)])


def _edge_call(pv, qc, pc, qv, ea_flat, wv, wc, zeros, src, dst,
               with_deg=False):
    nv = pv.shape[0]
    nc = pc.shape[0]
    n_pad = zeros.shape[0]
    e = src.shape[0]
    mesh = plsc.VectorSubcoreMesh(core_axis_name="c", subcore_axis_name="s")

    ns = _NSLOT

    nout = 4 if with_deg else 2

    def body(pv_h, qc_h, pc_h, qv_h, ea_h, wv_h, wc_h, z_h, src_h, dst_h,
             *rest):
        outs = rest[:nout]
        hv_h, hc_h = outs[0], outs[1]
        dv_h = outs[2] if with_deg else None
        dc_h = outs[3] if with_deg else None
        accum = rest[nout]
        scr = rest[nout + 1:]
        cid = lax.axis_index("c")
        sid = lax.axis_index("s")
        idx_o = list(scr[0:ns])
        idx_n = list(scr[ns:2 * ns])
        buf_p = list(scr[2 * ns:3 * ns])
        buf_q = list(scr[3 * ns:4 * ns])
        buf_e = list(scr[4 * ns:5 * ns])
        wbuf = scr[5 * ns]
        ones = scr[5 * ns + 1] if with_deg else None
        sems = scr[5 * ns + (2 if with_deg else 1):]
        sem_p = list(sems[0:ns])
        sem_q = list(sems[ns:2 * ns])
        sem_s = list(sems[2 * ns:3 * ns])
        sem_io = list(sems[3 * ns:4 * ns])
        sem_in = list(sems[4 * ns:5 * ns])
        sem_ea = list(sems[5 * ns:6 * ns])

        @pl.when(cid == 0)
        def _():
            _edge_side(pv_h, qc_h, ea_h, wv_h, z_h, src_h, dst_h, hv_h,
                       accum, idx_o, idx_n, buf_p, buf_q, buf_e, wbuf,
                       sem_p, sem_q, sem_s, sem_io, sem_in, sem_ea,
                       sid, e, n_pad, dv_h, ones)

        @pl.when(cid == 1)
        def _():
            _edge_side(pc_h, qv_h, ea_h, wc_h, z_h, dst_h, src_h, hc_h,
                       accum, idx_o, idx_n, buf_p, buf_q, buf_e, wbuf,
                       sem_p, sem_q, sem_s, sem_io, sem_in, sem_ea,
                       sid, e, n_pad, dc_h, ones)

    f = pl.kernel(
        body,
        out_type=[jax.ShapeDtypeStruct((n_pad, _F), jnp.float32)] * nout,
        mesh=mesh,
        scratch_types=(
            [pltpu.VMEM_SHARED((n_pad, _F), jnp.float32)]
            + [pltpu.VMEM((_CB,), jnp.int32)] * (2 * ns)
            + [pltpu.VMEM((_CB, _F), jnp.float32)] * (2 * ns)
            + [pltpu.VMEM((_CB * 4 + 16,), jnp.float32)] * ns
            + [pltpu.VMEM((4, _F), jnp.float32)]
            + ([pltpu.VMEM((_CB, _F), jnp.float32)] if with_deg else [])
            + [pltpu.SemaphoreType.DMA] * (6 * ns)
        ),
    )
    return f(pv, qc, pc, qv, ea_flat, wv, wc, zeros, src, dst)


def kernel(var_features, cons_features, edge_index, edge_attr,
           vmsg_W1, vmsg_b1, vmsg_W2, vmsg_b2,
           cmsg_W1, cmsg_b1, cmsg_W2, cmsg_b2,
           vupd_W1, vupd_b1, vupd_W2, vupd_b2,
           cupd_W1, cupd_b1, cupd_W2, cupd_b2,
           vbn_gamma, vbn_beta, cbn_gamma, cbn_beta):
    f = var_features.shape[1]
    num_iter = vmsg_W1.shape[0]
    src = edge_index[0]
    dst = edge_index[1]
    nv = var_features.shape[0]
    nc = cons_features.shape[0]
    n_pad = ((max(nv, nc) + _NSUB * 128 - 1) // (_NSUB * 128)) * (_NSUB * 128)

    ea_flat = edge_attr.reshape(-1)
    zeros = jnp.zeros((n_pad, _F), jnp.float32)

    vf, cf = var_features, cons_features
    pv, qc, pc, qv = _pre_call(
        vf, cf,
        vmsg_W1[0, :f], vmsg_W1[0, f:2 * f], vmsg_b1[0],
        cmsg_W1[0, :f], cmsg_W1[0, f:2 * f], cmsg_b1[0])
    for it in range(num_iter):
        if it == 0:
            hv, hc, degv, degc = _edge_call(
                pv, qc, pc, qv, ea_flat,
                vmsg_W1[it, 2 * f:], cmsg_W1[it, 2 * f:],
                zeros, src, dst, with_deg=True)
        else:
            hv, hc = _edge_call(pv, qc, pc, qv, ea_flat,
                                vmsg_W1[it, 2 * f:], cmsg_W1[it, 2 * f:],
                                zeros, src, dst)
        uv, sv1, sv2, uc, sc1, sc2 = _post1_call(
            hv, hc, degv, degc, vf, cf,
            vmsg_W2[it], vmsg_b2[it], cmsg_W2[it], cmsg_b2[it],
            vupd_W1[it, :f], vupd_W1[it, f:], vupd_b1[it],
            vupd_W2[it], vupd_b2[it],
            cupd_W1[it, :f], cupd_W1[it, f:], cupd_b1[it],
            cupd_W2[it], cupd_b2[it])
        if it + 1 < num_iter:
            nx = it + 1
            vf, cf, pv, qc, pc, qv = _bnpre_call(
                uv, sv1, sv2, vbn_gamma[it], vbn_beta[it],
                uc, sc1, sc2, cbn_gamma[it], cbn_beta[it],
                vmsg_W1[nx, :f], vmsg_W1[nx, f:2 * f], vmsg_b1[nx],
                cmsg_W1[nx, :f], cmsg_W1[nx, f:2 * f], cmsg_b1[nx])
        else:
            vf, cf = _bn2_call(
                uv, sv1, sv2, vbn_gamma[it], vbn_beta[it],
                uc, sc1, sc2, cbn_gamma[it], cbn_beta[it])
    return vf, cf


# speedup vs baseline: 1.0466x; 1.0427x over previous
import jax
import jax.numpy as jnp
from jax import lax
from jax.experimental import pallas as pl
from jax.experimental.pallas import tpu as pltpu
from jax.experimental.pallas import tpu_sc as plsc

_EPS = 1e-5
_F = 128
_RB = 2000
_CB = 40
_NSLOT = 4
_NSUB = 16


def _pre_body(vf, cf, wav, wbv, b1v, wac, wbc, b1c, pv, qc, pc, qv):
    v = vf[...]
    c = cf[...]
    pv[...] = jnp.dot(v, wav[...], preferred_element_type=jnp.float32, precision=lax.Precision.HIGHEST) + b1v[...]
    qc[...] = jnp.dot(c, wbv[...], preferred_element_type=jnp.float32, precision=lax.Precision.HIGHEST)
    pc[...] = jnp.dot(c, wac[...], preferred_element_type=jnp.float32, precision=lax.Precision.HIGHEST) + b1c[...]
    qv[...] = jnp.dot(v, wbc[...], preferred_element_type=jnp.float32, precision=lax.Precision.HIGHEST)


def _pre_call(vf, cf, wav, wbv, b1v, wac, wbc, b1c):
    n = vf.shape[0]
    grid = (n // _RB,)
    blk = pl.BlockSpec((_RB, _F), lambda i: (i, 0))
    wblk = pl.BlockSpec((_F, _F), lambda i: (0, 0))
    bblk = pl.BlockSpec((1, _F), lambda i: (0, 0))
    out = jax.ShapeDtypeStruct((n, _F), jnp.float32)
    return pl.pallas_call(
        _pre_body,
        grid=grid,
        in_specs=[blk, blk, wblk, wblk, bblk, wblk, wblk, bblk],
        out_specs=[blk, blk, blk, blk],
        out_shape=[out, out, out, out],
    )(vf, cf, wav, wbv, b1v[None, :], wac, wbc, b1c[None, :])


def _post1_body(hv, hc, degv, degc, vf, cf,
                w2v, b2v, w2c, b2c,
                uav, ubv, b1uv, w2uv, b2uv,
                uac, ubc, b1uc, w2uc, b2uc,
                uv, sv1, sv2, uc, sc1, sc2):
    aggv = (jnp.dot(hv[...], w2v[...], preferred_element_type=jnp.float32, precision=lax.Precision.HIGHEST)
            + degv[...][:, 0:1] * b2v[...])
    x = (jnp.dot(vf[...], uav[...], preferred_element_type=jnp.float32, precision=lax.Precision.HIGHEST)
         + jnp.dot(aggv, ubv[...], preferred_element_type=jnp.float32, precision=lax.Precision.HIGHEST)
         + b1uv[...])
    u = (jnp.dot(jnp.maximum(x, 0.0), w2uv[...],
                 preferred_element_type=jnp.float32, precision=lax.Precision.HIGHEST) + b2uv[...])
    uv[...] = u
    sv1[...] = jnp.sum(u, axis=0).reshape(1, 1, _F)
    sv2[...] = jnp.sum(u * u, axis=0).reshape(1, 1, _F)

    aggc = (jnp.dot(hc[...], w2c[...], preferred_element_type=jnp.float32, precision=lax.Precision.HIGHEST)
            + degc[...][:, 0:1] * b2c[...])
    y = (jnp.dot(cf[...], uac[...], preferred_element_type=jnp.float32, precision=lax.Precision.HIGHEST)
         + jnp.dot(aggc, ubc[...], preferred_element_type=jnp.float32, precision=lax.Precision.HIGHEST)
         + b1uc[...])
    w = (jnp.dot(jnp.maximum(y, 0.0), w2uc[...],
                 preferred_element_type=jnp.float32, precision=lax.Precision.HIGHEST) + b2uc[...])
    uc[...] = w
    sc1[...] = jnp.sum(w, axis=0).reshape(1, 1, _F)
    sc2[...] = jnp.sum(w * w, axis=0).reshape(1, 1, _F)


def _post1_call(hv, hc, degv, degc, vf, cf, w2v, b2v, w2c, b2c,
                uav, ubv, b1uv, w2uv, b2uv, uac, ubc, b1uc, w2uc, b2uc):
    n = vf.shape[0]
    nb = n // _RB
    grid = (nb,)
    blk = pl.BlockSpec((_RB, _F), lambda i: (i, 0))
    dblk = pl.BlockSpec((_RB, _F), lambda i: (i, 0))
    wblk = pl.BlockSpec((_F, _F), lambda i: (0, 0))
    bblk = pl.BlockSpec((1, _F), lambda i: (0, 0))
    sblk = pl.BlockSpec((1, 1, _F), lambda i: (i, 0, 0))
    outn = jax.ShapeDtypeStruct((n, _F), jnp.float32)
    outs = jax.ShapeDtypeStruct((nb, 1, _F), jnp.float32)
    return pl.pallas_call(
        _post1_body,
        grid=grid,
        in_specs=[blk, blk, dblk, dblk, blk, blk,
                  wblk, bblk, wblk, bblk,
                  wblk, wblk, bblk, wblk, bblk,
                  wblk, wblk, bblk, wblk, bblk],
        out_specs=[blk, sblk, sblk, blk, sblk, sblk],
        out_shape=[outn, outs, outs, outn, outs, outs],
    )(hv, hc, degv, degc, vf, cf,
      w2v, b2v[None, :], w2c, b2c[None, :],
      uav, ubv, b1uv[None, :], w2uv, b2uv[None, :],
      uac, ubc, b1uc[None, :], w2uc, b2uc[None, :])


def _bn_expr(u, s1, s2, g, b, n):
    m = jnp.sum(s1[...], axis=0) / float(n)
    var = jnp.sum(s2[...], axis=0) / float(n) - m * m
    return g[...] * (u[...] - m) * lax.rsqrt(var + _EPS) + b[...]


def _bn2_call(uv, sv1, sv2, vg, vb, uc, sc1, sc2, cg, cb):
    n = uv.shape[0]
    nb = n // _RB

    def body(uv, sv1, sv2, vg, vb, uc, sc1, sc2, cg, cb, ov, oc):
        ov[...] = _bn_expr(uv, sv1, sv2, vg, vb, n)
        oc[...] = _bn_expr(uc, sc1, sc2, cg, cb, n)

    blk = pl.BlockSpec((_RB, _F), lambda i: (i, 0))
    sfull = pl.BlockSpec((nb, 1, _F), lambda i: (0, 0, 0))
    bblk = pl.BlockSpec((1, _F), lambda i: (0, 0))
    out = jax.ShapeDtypeStruct((n, _F), jnp.float32)
    return pl.pallas_call(
        body,
        grid=(nb,),
        in_specs=[blk, sfull, sfull, bblk, bblk, blk, sfull, sfull, bblk, bblk],
        out_specs=[blk, blk],
        out_shape=[out, out],
    )(uv, sv1, sv2, vg[None, :], vb[None, :],
      uc, sc1, sc2, cg[None, :], cb[None, :])


def _bnpre_call(uv, sv1, sv2, vg, vb, uc, sc1, sc2, cg, cb,
                wav, wbv, b1v, wac, wbc, b1c):
    n = uv.shape[0]
    nb = n // _RB

    def body(uv, sv1, sv2, vg, vb, uc, sc1, sc2, cg, cb,
             wav, wbv, b1v, wac, wbc, b1c,
             ov, oc, pv, qc, pc, qv):
        v = _bn_expr(uv, sv1, sv2, vg, vb, n)
        c = _bn_expr(uc, sc1, sc2, cg, cb, n)
        ov[...] = v
        oc[...] = c
        pv[...] = jnp.dot(v, wav[...], preferred_element_type=jnp.float32,
                          precision=lax.Precision.HIGHEST) + b1v[...]
        qc[...] = jnp.dot(c, wbv[...], preferred_element_type=jnp.float32,
                          precision=lax.Precision.HIGHEST)
        pc[...] = jnp.dot(c, wac[...], preferred_element_type=jnp.float32,
                          precision=lax.Precision.HIGHEST) + b1c[...]
        qv[...] = jnp.dot(v, wbc[...], preferred_element_type=jnp.float32,
                          precision=lax.Precision.HIGHEST)

    blk = pl.BlockSpec((_RB, _F), lambda i: (i, 0))
    sfull = pl.BlockSpec((nb, 1, _F), lambda i: (0, 0, 0))
    bblk = pl.BlockSpec((1, _F), lambda i: (0, 0))
    wblk = pl.BlockSpec((_F, _F), lambda i: (0, 0))
    out = jax.ShapeDtypeStruct((n, _F), jnp.float32)
    return pl.pallas_call(
        body,
        grid=(nb,),
        in_specs=[blk, sfull, sfull, bblk, bblk, blk, sfull, sfull, bblk, bblk,
                  wblk, wblk, bblk, wblk, wblk, bblk],
        out_specs=[blk] * 6,
        out_shape=[out] * 6,
    )(uv, sv1, sv2, vg[None, :], vb[None, :],
      uc, sc1, sc2, cg[None, :], cb[None, :],
      wav, wbv, b1v[None, :], wac, wbc, b1c[None, :])


def _edge_side(p_hbm, q_hbm, ea_hbm, w_hbm, z_hbm, own_hbm, nbr_hbm, out_hbm,
               accum, idx_o, idx_n, buf_p, buf_q, buf_e, wbuf, sem_p, sem_q,
               sem_s, sem_io, sem_in, sem_ea, sid, e_total, n_pad,
               deg_out_hbm=None, ones=None):
    rows_per_sub = n_pad // _NSUB
    rsl = pl.ds(sid * rows_per_sub, rows_per_sub)
    pltpu.sync_copy(z_hbm.at[rsl], accum.at[rsl])
    pltpu.sync_copy(w_hbm, wbuf)
    plsc.subcore_barrier()

    if deg_out_hbm is not None:
        nch_d = e_total // _CB // _NSUB
        one = jnp.ones((16,), jnp.float32)

        def orow(r, c):
            for j in range(_F // 16):
                ones[r, pl.ds(j * 16, 16)] = one
            return c

        lax.fori_loop(0, _CB, orow, 0)

        def d_issue(b, ci):
            base = (ci * _NSUB + sid) * _CB
            pltpu.async_copy(own_hbm.at[pl.ds(base, _CB)], idx_o[b],
                             sem_io[b])

        def d_wait_idx(b):
            pltpu.make_async_copy(own_hbm.at[pl.ds(0, _CB)], idx_o[b],
                                  sem_io[b]).wait()

        def d_wait_sc(b):
            pltpu.make_async_copy(ones, accum.at[idx_o[b]], sem_s[b]).wait()

        d_issue(0, 0)
        d_issue(1, 1)
        d_issue(2, 2)

        def dgroup(g, carry):
            for b in range(_NSLOT):
                ci = g * _NSLOT + b
                d_wait_idx(b)
                pltpu.async_copy(ones, accum.at[idx_o[b]], sem_s[b],
                                 add=True)
                s3 = (b + 3) % _NSLOT

                @pl.when(jnp.logical_and(ci >= 1, ci + 3 < nch_d))
                def _():
                    d_wait_sc(s3)

                @pl.when(ci + 3 < nch_d)
                def _():
                    d_issue(s3, ci + 3)
            return carry

        lax.fori_loop(0, nch_d // _NSLOT, dgroup, 0)
        for b in range(_NSLOT):
            d_wait_sc((nch_d - 4 + b) % _NSLOT)
        plsc.subcore_barrier()
        pltpu.sync_copy(accum.at[rsl], deg_out_hbm.at[rsl])
        pltpu.sync_copy(z_hbm.at[rsl], accum.at[rsl])
        plsc.subcore_barrier()

    w_vals = [[wbuf[k, pl.ds(j * 16, 16)] for j in range(_F // 16)]
              for k in range(4)]

    nch = e_total // _CB // _NSUB

    def issue_idx(b, ci):
        base = (ci * _NSUB + sid) * _CB
        pltpu.async_copy(own_hbm.at[pl.ds(base, _CB)], idx_o[b], sem_io[b])
        pltpu.async_copy(nbr_hbm.at[pl.ds(base, _CB)], idx_n[b], sem_in[b])
        pltpu.async_copy(ea_hbm.at[pl.ds(base * 4, _CB * 4)],
                         buf_e[b].at[pl.ds(0, _CB * 4)], sem_ea[b])

    def wait_idx(b):
        base = 0
        pltpu.make_async_copy(own_hbm.at[pl.ds(base, _CB)], idx_o[b],
                              sem_io[b]).wait()
        pltpu.make_async_copy(nbr_hbm.at[pl.ds(base, _CB)], idx_n[b],
                              sem_in[b]).wait()
        pltpu.make_async_copy(ea_hbm.at[pl.ds(base * 4, _CB * 4)],
                              buf_e[b].at[pl.ds(0, _CB * 4)],
                              sem_ea[b]).wait()

    def issue_gather(b):
        pltpu.async_copy(p_hbm.at[idx_o[b]], buf_p[b], sem_p[b])
        pltpu.async_copy(q_hbm.at[idx_n[b]], buf_q[b], sem_q[b])

    def wait_scatter(b):
        pltpu.make_async_copy(buf_p[b], accum.at[idx_o[b]], sem_s[b]).wait()

    def compute(b):
        pltpu.make_async_copy(p_hbm.at[idx_o[b]], buf_p[b], sem_p[b]).wait()
        pltpu.make_async_copy(q_hbm.at[idx_n[b]], buf_q[b], sem_q[b]).wait()
        bp, bq, eb = buf_p[b], buf_q[b], buf_e[b]

        def row(rr, c2):
            av = eb[pl.ds(rr * 8, 16)]
            for h in range(2):
                r = rr * 2 + h
                a0 = av[4 * h]
                a1 = av[4 * h + 1]
                a2 = av[4 * h + 2]
                a3 = av[4 * h + 3]
                for j in range(_F // 16):
                    s = pl.ds(j * 16, 16)
                    acc = bp[r, s] + bq[r, s]
                    acc = acc + a0 * w_vals[0][j]
                    acc = acc + a1 * w_vals[1][j]
                    acc = acc + a2 * w_vals[2][j]
                    acc = acc + a3 * w_vals[3][j]
                    bp[r, s] = jnp.maximum(acc, 0.0)
            return c2

        lax.fori_loop(0, _CB // 2, row, 0)
        pltpu.async_copy(bp, accum.at[idx_o[b]], sem_s[b], add=True)

    issue_idx(0, 0)
    issue_idx(1, 1)
    issue_idx(2, 2)
    wait_idx(0)
    issue_gather(0)
    wait_idx(1)
    issue_gather(1)

    def group(g, carry):
        for b in range(_NSLOT):
            ci = g * _NSLOT + b
            compute(b)
            s3 = (b + 3) % _NSLOT

            @pl.when(jnp.logical_and(ci >= 1, ci + 3 < nch))
            def _():
                wait_scatter(s3)

            @pl.when(ci + 3 < nch)
            def _():
                issue_idx(s3, ci + 3)

            s2 = (b + 2) % _NSLOT

            @pl.when(ci + 2 < nch)
            def _():
                wait_idx(s2)
                issue_gather(s2)
        return carry

    lax.fori_loop(0, nch // _NSLOT, group, 0)
    for b in range(_NSLOT):
        wait_scatter((nch - 4 + b) % _NSLOT)
    plsc.subcore_barrier()
    pltpu.sync_copy(accum.at[pl.ds(sid * rows_per_sub, rows_per_sub)],
                    out_hbm.at[pl.ds(sid * rows_per_sub, rows_per_sub)])


def _edge_call(pv, qc, pc, qv, ea_flat, wv, wc, zeros, src, dst,
               with_deg=False):
    nv = pv.shape[0]
    nc = pc.shape[0]
    n_pad = zeros.shape[0]
    e = src.shape[0]
    mesh = plsc.VectorSubcoreMesh(core_axis_name="c", subcore_axis_name="s")

    ns = _NSLOT

    nout = 4 if with_deg else 2

    def body(pv_h, qc_h, pc_h, qv_h, ea_h, wv_h, wc_h, z_h, src_h, dst_h,
             *rest):
        outs = rest[:nout]
        hv_h, hc_h = outs[0], outs[1]
        dv_h = outs[2] if with_deg else None
        dc_h = outs[3] if with_deg else None
        accum = rest[nout]
        scr = rest[nout + 1:]
        cid = lax.axis_index("c")
        sid = lax.axis_index("s")
        idx_o = list(scr[0:ns])
        idx_n = list(scr[ns:2 * ns])
        buf_p = list(scr[2 * ns:3 * ns])
        buf_q = list(scr[3 * ns:4 * ns])
        buf_e = list(scr[4 * ns:5 * ns])
        wbuf = scr[5 * ns]
        ones = scr[5 * ns + 1] if with_deg else None
        sems = scr[5 * ns + (2 if with_deg else 1):]
        sem_p = list(sems[0:ns])
        sem_q = list(sems[ns:2 * ns])
        sem_s = list(sems[2 * ns:3 * ns])
        sem_io = list(sems[3 * ns:4 * ns])
        sem_in = list(sems[4 * ns:5 * ns])
        sem_ea = list(sems[5 * ns:6 * ns])

        @pl.when(cid == 0)
        def _():
            _edge_side(pv_h, qc_h, ea_h, wv_h, z_h, src_h, dst_h, hv_h,
                       accum, idx_o, idx_n, buf_p, buf_q, buf_e, wbuf,
                       sem_p, sem_q, sem_s, sem_io, sem_in, sem_ea,
                       sid, e, n_pad, dv_h, ones)

        @pl.when(cid == 1)
        def _():
            _edge_side(pc_h, qv_h, ea_h, wc_h, z_h, dst_h, src_h, hc_h,
                       accum, idx_o, idx_n, buf_p, buf_q, buf_e, wbuf,
                       sem_p, sem_q, sem_s, sem_io, sem_in, sem_ea,
                       sid, e, n_pad, dc_h, ones)

    f = pl.kernel(
        body,
        out_type=[jax.ShapeDtypeStruct((n_pad, _F), jnp.float32)] * nout,
        mesh=mesh,
        scratch_types=(
            [pltpu.VMEM_SHARED((n_pad, _F), jnp.float32)]
            + [pltpu.VMEM((_CB,), jnp.int32)] * (2 * ns)
            + [pltpu.VMEM((_CB, _F), jnp.float32)] * (2 * ns)
            + [pltpu.VMEM((_CB * 4 + 16,), jnp.float32)] * ns
            + [pltpu.VMEM((4, _F), jnp.float32)]
            + ([pltpu.VMEM((_CB, _F), jnp.float32)] if with_deg else [])
            + [pltpu.SemaphoreType.DMA] * (6 * ns)
        ),
    )
    return f(pv, qc, pc, qv, ea_flat, wv, wc, zeros, src, dst)


def kernel(var_features, cons_features, edge_index, edge_attr,
           vmsg_W1, vmsg_b1, vmsg_W2, vmsg_b2,
           cmsg_W1, cmsg_b1, cmsg_W2, cmsg_b2,
           vupd_W1, vupd_b1, vupd_W2, vupd_b2,
           cupd_W1, cupd_b1, cupd_W2, cupd_b2,
           vbn_gamma, vbn_beta, cbn_gamma, cbn_beta):
    f = var_features.shape[1]
    num_iter = vmsg_W1.shape[0]
    src = edge_index[0]
    dst = edge_index[1]
    nv = var_features.shape[0]
    nc = cons_features.shape[0]
    n_pad = ((max(nv, nc) + _NSUB * 128 - 1) // (_NSUB * 128)) * (_NSUB * 128)

    ea_flat = edge_attr.reshape(-1)
    zeros = jnp.zeros((n_pad, _F), jnp.float32)

    vf, cf = var_features, cons_features
    pv, qc, pc, qv = _pre_call(
        vf, cf,
        vmsg_W1[0, :f], vmsg_W1[0, f:2 * f], vmsg_b1[0],
        cmsg_W1[0, :f], cmsg_W1[0, f:2 * f], cmsg_b1[0])
    for it in range(num_iter):
        if it == 0:
            hv, hc, degv, degc = _edge_call(
                pv, qc, pc, qv, ea_flat,
                vmsg_W1[it, 2 * f:], cmsg_W1[it, 2 * f:],
                zeros, src, dst, with_deg=True)
        else:
            hv, hc = _edge_call(pv, qc, pc, qv, ea_flat,
                                vmsg_W1[it, 2 * f:], cmsg_W1[it, 2 * f:],
                                zeros, src, dst)
        uv, sv1, sv2, uc, sc1, sc2 = _post1_call(
            hv, hc, degv, degc, vf, cf,
            vmsg_W2[it], vmsg_b2[it], cmsg_W2[it], cmsg_b2[it],
            vupd_W1[it, :f], vupd_W1[it, f:], vupd_b1[it],
            vupd_W2[it], vupd_b2[it],
            cupd_W1[it, :f], cupd_W1[it, f:], cupd_b1[it],
            cupd_W2[it], cupd_b2[it])
        if it + 1 < num_iter:
            nx = it + 1
            vf, cf, pv, qc, pc, qv = _bnpre_call(
                uv, sv1, sv2, vbn_gamma[it], vbn_beta[it],
                uc, sc1, sc2, cbn_gamma[it], cbn_beta[it],
                vmsg_W1[nx, :f], vmsg_W1[nx, f:2 * f], vmsg_b1[nx],
                cmsg_W1[nx, :f], cmsg_W1[nx, f:2 * f], cmsg_b1[nx])
        else:
            vf, cf = _bn2_call(
                uv, sv1, sv2, vbn_gamma[it], vbn_beta[it],
                uc, sc1, sc2, cbn_gamma[it], cbn_beta[it])
    return vf, cf
